# baseline (speedup 1.0000x reference)
import functools

import numpy as np

import jax
import jax.numpy as jnp
from jax import lax
from jax.experimental import pallas as pl
from jax.experimental.pallas import tpu as pltpu
from jax.experimental.pallas import tpu_sc as plsc

N_NODES = 10000
N_EDGES = 320000
D_IN = 128
D_HID = 64
D_OUT = 16

N_PAD = 10240
NCORE = 2
NSUB = 16
NW = NCORE * NSUB
CHUNK = 128
PAD_CHUNKS = 8


def _make_spmm(D, n_chunks, nbuf, n_phases, h_dtype=jnp.float32):
  rows_per_sub = N_PAD // NSUB
  h_rows_per_sub = N_NODES // NSUB
  ph_chunks = n_chunks // n_phases
  mesh = plsc.VectorSubcoreMesh(core_axis_name="c", subcore_axis_name="s")

  @functools.partial(
      pl.kernel,
      out_type=jax.ShapeDtypeStruct((NCORE, N_PAD, D), jnp.float32),
      mesh=mesh,
      scratch_types=[
          pltpu.VMEM((ph_chunks, CHUNK), jnp.int32),
          pltpu.VMEM((ph_chunks, CHUNK), jnp.int32),
          pltpu.VMEM((ph_chunks, CHUNK), jnp.float32),
          [pltpu.VMEM((CHUNK, D), h_dtype)] * nbuf,
          [pltpu.VMEM((CHUNK, D), jnp.float32)] * nbuf,
          [pltpu.SemaphoreType.DMA] * nbuf,
          [pltpu.SemaphoreType.DMA] * nbuf,
          pltpu.VMEM_SHARED((N_NODES, D), h_dtype),
          pltpu.VMEM_SHARED((N_PAD, D), jnp.float32),
      ],
      compiler_params=pltpu.CompilerParams(use_tc_tiling_on_sc=False,
                                           needs_layout_passes=False),
  )
  def spmm(h_hbm, src_hbm, dst_hbm, w_hbm, zeros_hbm, out_hbm,
           src_v, dst_v, w_v, grows, srows, gsem, ssem, h_sp, acc):
    c = lax.axis_index("c")
    s = lax.axis_index("s")
    wid = s * NCORE + c
    hsl = pl.ds(s * h_rows_per_sub, h_rows_per_sub)
    pltpu.sync_copy(h_hbm.at[hsl], h_sp.at[hsl])
    pltpu.sync_copy(zeros_hbm, acc.at[pl.ds(s * rows_per_sub, rows_per_sub)])
    plsc.subcore_barrier()

    for ph in range(n_phases):
      psl = pl.ds(ph * ph_chunks, ph_chunks)
      pltpu.sync_copy(src_hbm.at[wid, psl], src_v)
      pltpu.sync_copy(dst_hbm.at[wid, psl], dst_v)
      pltpu.sync_copy(w_hbm.at[wid, psl], w_v)


      for b in range(nbuf):
        pltpu.async_copy(h_sp.at[src_v.at[b]], grows[b], gsem[b])

      @pl.loop(0, ph_chunks // nbuf)
      def _(i):
        for b in range(nbuf):
          ci = i * nbuf + b
          pltpu.make_async_copy(
              h_sp.at[src_v.at[ci]], grows[b], gsem[b]).wait()

          @pl.when(i > 0)
          def _():
            pltpu.make_async_copy(
                srows[b], acc.at[dst_v.at[ci]], ssem[b]).wait()

          @pl.loop(0, CHUNK // 16)
          def _(g):
            w16 = w_v[ci, pl.ds(g * 16, 16)]
            base = g * 16
            for e in range(16):
              wv = w16[e]
              if h_dtype == jnp.bfloat16:
                for j in range(D // 32):
                  pair = grows[b][base + e, pl.ds(j * 32, 32)]
                  u = plsc.bitcast(pair, jnp.uint32)
                  lo = plsc.bitcast(u << jnp.uint32(16), jnp.float32)
                  hi = plsc.bitcast(u & jnp.uint32(0xFFFF0000), jnp.float32)
                  srows[b].at[base + e, pl.ds(j * 32, 16)][...] = lo * wv
                  srows[b].at[base + e, pl.ds(j * 32 + 16, 16)][...] = hi * wv
              else:
                for j in range(D // 16):
                  sl = (base + e, pl.ds(j * 16, 16))
                  srows[b].at[sl][...] = grows[b].at[sl][...] * wv

          pltpu.async_copy(srows[b], acc.at[dst_v.at[ci]], ssem[b], add=True)

          @pl.when(ci + nbuf < ph_chunks)
          def _():
            nci = ci + nbuf
            pltpu.async_copy(h_sp.at[src_v.at[nci]], grows[b], gsem[b])

      for b in range(nbuf):
        pltpu.make_async_copy(srows[b], acc.at[dst_v.at[0]], ssem[b]).wait()

    plsc.subcore_barrier()
    rsl = pl.ds(s * rows_per_sub, rows_per_sub)
    pltpu.sync_copy(acc.at[rsl], out_hbm.at[c, rsl])

  return spmm


def _mm1(x, W0):
  def body(x_ref, w_ref, o_ref):
    o_ref[...] = jnp.dot(x_ref[...], w_ref[...],
                         preferred_element_type=jnp.float32
                         ).astype(jnp.bfloat16)

  return pl.pallas_call(
      body,
      out_shape=jax.ShapeDtypeStruct((N_NODES, D_HID), jnp.bfloat16),
      grid=(5,),
      in_specs=[
          pl.BlockSpec((2000, D_IN), lambda i: (i, 0)),
          pl.BlockSpec((D_IN, D_HID), lambda i: (0, 0)),
      ],
      out_specs=pl.BlockSpec((2000, D_HID), lambda i: (i, 0)),
  )(x, W0)


def _combine_relu_mm(p, W1):
  def body(p_ref, w_ref, o_ref):
    h = jnp.maximum(p_ref[0] + p_ref[1], 0.0)
    o_ref[...] = jnp.dot(h, w_ref[...], preferred_element_type=jnp.float32)

  return pl.pallas_call(
      body,
      out_shape=jax.ShapeDtypeStruct((N_NODES, D_OUT), jnp.float32),
      grid=(5,),
      in_specs=[
          pl.BlockSpec((NCORE, 2000, D_HID), lambda i: (0, i, 0)),
          pl.BlockSpec((D_HID, D_OUT), lambda i: (0, 0)),
      ],
      out_specs=pl.BlockSpec((2000, D_OUT), lambda i: (i, 0)),
  )(p, W1)


def _combine_softmax(p):
  def body(p_ref, o_ref):
    h = p_ref[0] + p_ref[1]
    m = jnp.max(h, axis=1, keepdims=True)
    e = jnp.exp(h - m)
    o_ref[...] = e / jnp.sum(e, axis=1, keepdims=True)

  return pl.pallas_call(
      body,
      out_shape=jax.ShapeDtypeStruct((N_NODES, D_OUT), jnp.float32),
      grid=(5,),
      in_specs=[pl.BlockSpec((NCORE, 2000, D_OUT), lambda i: (0, i, 0))],
      out_specs=pl.BlockSpec((2000, D_OUT), lambda i: (i, 0)),
  )(p)


def kernel(x, edge_index, edge_weight, W0, W1):
  step = PAD_CHUNKS * CHUNK
  per_w = -(-N_EDGES // (NW * step)) * step
  n_chunks = per_w // CHUNK
  e_pad = NW * per_w - N_EDGES

  src = edge_index[0].astype(jnp.int32)
  dst = edge_index[1].astype(jnp.int32)
  w = edge_weight.astype(jnp.float32)
  src = jnp.concatenate([src, jnp.zeros((e_pad,), jnp.int32)])
  dst = jnp.concatenate([dst, jnp.zeros((e_pad,), jnp.int32)])
  w = jnp.concatenate([w, jnp.zeros((e_pad,), jnp.float32)])
  src_r = src.reshape(NW, n_chunks, CHUNK)
  dst_r = dst.reshape(NW, n_chunks, CHUNK)
  w_r = w.reshape(NW, n_chunks, CHUNK)

  spmm_hid = _make_spmm(D_HID, n_chunks, 4, 2, h_dtype=jnp.bfloat16)
  spmm_out = _make_spmm(D_OUT, n_chunks, 4, 1)
  zeros_hid = jnp.zeros((N_PAD // NSUB, D_HID), jnp.float32)
  zeros_out = jnp.zeros((N_PAD // NSUB, D_OUT), jnp.float32)

  pos_f = np.array([32 * j + 2 * q + hh
                    for j in range(D_HID // 32)
                    for hh in range(2)
                    for q in range(16)])
  W1p = W1[pos_f, :]

  h = _mm1(x, W0)
  p1 = spmm_hid(h, src_r, dst_r, w_r, zeros_hid)
  h1 = _combine_relu_mm(p1, W1p)
  p2 = spmm_out(h1, src_r, dst_r, w_r, zeros_out)
  return _combine_softmax(p2)


# speedup vs baseline: 1.2636x; 1.2636x over previous
import functools

import jax
import jax.numpy as jnp
from jax import lax
from jax.experimental import pallas as pl
from jax.experimental.pallas import tpu as pltpu
from jax.experimental.pallas import tpu_sc as plsc

N_NODES = 10000
N_EDGES = 320000
D_IN = 128
D_HID = 64
D_OUT = 16

N_PAD = 10240
NCORE = 2
NSUB = 16
NW = NCORE * NSUB
CHUNK = 128
PAD_CHUNKS = 8


def _make_spmm(D, n_chunks, nbuf, n_phases):
  rows_per_sub = N_PAD // NSUB
  h_rows_per_sub = N_NODES // NSUB
  ph_chunks = n_chunks // n_phases
  mesh = plsc.VectorSubcoreMesh(core_axis_name="c", subcore_axis_name="s")

  @functools.partial(
      pl.kernel,
      out_type=jax.ShapeDtypeStruct((NCORE, N_PAD, D), jnp.float32),
      mesh=mesh,
      scratch_types=[
          pltpu.VMEM((ph_chunks, CHUNK), jnp.int32),
          pltpu.VMEM((ph_chunks, CHUNK), jnp.int32),
          pltpu.VMEM((ph_chunks, CHUNK), jnp.float32),
          [pltpu.VMEM((CHUNK, D), jnp.float32)] * nbuf,
          [pltpu.VMEM((CHUNK, D), jnp.float32)] * nbuf,
          [pltpu.SemaphoreType.DMA] * nbuf,
          [pltpu.SemaphoreType.DMA] * nbuf,
          pltpu.VMEM_SHARED((N_NODES, D), jnp.float32),
          pltpu.VMEM_SHARED((N_PAD, D), jnp.float32),
      ],
      compiler_params=pltpu.CompilerParams(use_tc_tiling_on_sc=False),
  )
  def spmm(h_hbm, src_hbm, dst_hbm, w_hbm, zeros_hbm, out_hbm,
           src_v, dst_v, w_v, grows, srows, gsem, ssem, h_sp, acc):
    c = lax.axis_index("c")
    s = lax.axis_index("s")
    wid = s * NCORE + c
    hsl = pl.ds(s * h_rows_per_sub, h_rows_per_sub)
    pltpu.sync_copy(h_hbm.at[hsl], h_sp.at[hsl])
    pltpu.sync_copy(zeros_hbm, acc.at[pl.ds(s * rows_per_sub, rows_per_sub)])
    plsc.subcore_barrier()

    for ph in range(n_phases):
      psl = pl.ds(ph * ph_chunks, ph_chunks)
      pltpu.sync_copy(src_hbm.at[wid, psl], src_v)
      pltpu.sync_copy(dst_hbm.at[wid, psl], dst_v)
      pltpu.sync_copy(w_hbm.at[wid, psl], w_v)


      for b in range(nbuf):
        pltpu.async_copy(h_sp.at[src_v.at[b]], grows[b], gsem[b])

      @pl.loop(0, ph_chunks // nbuf)
      def _(i):
        for b in range(nbuf):
          ci = i * nbuf + b
          pltpu.make_async_copy(
              h_sp.at[src_v.at[ci]], grows[b], gsem[b]).wait()

          @pl.when(i > 0)
          def _():
            pltpu.make_async_copy(
                srows[b], acc.at[dst_v.at[ci]], ssem[b]).wait()

          @pl.loop(0, CHUNK // 16)
          def _(g):
            w16 = w_v[ci, pl.ds(g * 16, 16)]
            base = g * 16
            for e in range(16):
              wv = w16[e]
              for j in range(D // 16):
                sl = (base + e, pl.ds(j * 16, 16))
                srows[b].at[sl][...] = grows[b].at[sl][...] * wv

          pltpu.async_copy(srows[b], acc.at[dst_v.at[ci]], ssem[b], add=True)

          @pl.when(ci + nbuf < ph_chunks)
          def _():
            nci = ci + nbuf
            pltpu.async_copy(h_sp.at[src_v.at[nci]], grows[b], gsem[b])

      for b in range(nbuf):
        pltpu.make_async_copy(srows[b], acc.at[dst_v.at[0]], ssem[b]).wait()

    plsc.subcore_barrier()
    rsl = pl.ds(s * rows_per_sub, rows_per_sub)
    pltpu.sync_copy(acc.at[rsl], out_hbm.at[c, rsl])

  return spmm


def _mm1(x, W0):
  def body(x_ref, w_ref, o_ref):
    o_ref[...] = jnp.dot(x_ref[...], w_ref[...],
                         preferred_element_type=jnp.float32)

  return pl.pallas_call(
      body,
      out_shape=jax.ShapeDtypeStruct((N_NODES, D_HID), jnp.float32),
      grid=(1,),
      in_specs=[
          pl.BlockSpec((N_NODES, D_IN), lambda i: (0, 0)),
          pl.BlockSpec((D_IN, D_HID), lambda i: (0, 0)),
      ],
      out_specs=pl.BlockSpec((N_NODES, D_HID), lambda i: (0, 0)),
  )(x, W0)


def _combine_relu_mm(p, W1):
  def body(p_ref, w_ref, o_ref):
    h = jnp.maximum(p_ref[0] + p_ref[1], 0.0)
    o_ref[...] = jnp.dot(h, w_ref[...], preferred_element_type=jnp.float32)

  return pl.pallas_call(
      body,
      out_shape=jax.ShapeDtypeStruct((N_NODES, D_OUT), jnp.float32),
      grid=(1,),
      in_specs=[
          pl.BlockSpec((NCORE, N_NODES, D_HID), lambda i: (0, 0, 0)),
          pl.BlockSpec((D_HID, D_OUT), lambda i: (0, 0)),
      ],
      out_specs=pl.BlockSpec((N_NODES, D_OUT), lambda i: (0, 0)),
  )(p, W1)


def _combine_softmax(p):
  def body(p_ref, o_ref):
    h = p_ref[0] + p_ref[1]
    m = jnp.max(h, axis=1, keepdims=True)
    e = jnp.exp(h - m)
    o_ref[...] = e / jnp.sum(e, axis=1, keepdims=True)

  return pl.pallas_call(
      body,
      out_shape=jax.ShapeDtypeStruct((N_NODES, D_OUT), jnp.float32),
      grid=(1,),
      in_specs=[pl.BlockSpec((NCORE, N_NODES, D_OUT), lambda i: (0, 0, 0))],
      out_specs=pl.BlockSpec((N_NODES, D_OUT), lambda i: (0, 0)),
  )(p)


def kernel(x, edge_index, edge_weight, W0, W1):
  step = PAD_CHUNKS * CHUNK
  per_w = -(-N_EDGES // (NW * step)) * step
  n_chunks = per_w // CHUNK
  e_pad = NW * per_w - N_EDGES

  src = edge_index[0].astype(jnp.int32)
  dst = edge_index[1].astype(jnp.int32)
  w = edge_weight.astype(jnp.float32)
  src = jnp.concatenate([src, jnp.zeros((e_pad,), jnp.int32)])
  dst = jnp.concatenate([dst, jnp.zeros((e_pad,), jnp.int32)])
  w = jnp.concatenate([w, jnp.zeros((e_pad,), jnp.float32)])
  src_r = src.reshape(NW, n_chunks, CHUNK)
  dst_r = dst.reshape(NW, n_chunks, CHUNK)
  w_r = w.reshape(NW, n_chunks, CHUNK)

  spmm_hid = _make_spmm(D_HID, n_chunks, 2, 2)
  spmm_out = _make_spmm(D_OUT, n_chunks, 4, 1)
  zeros_hid = jnp.zeros((N_PAD // NSUB, D_HID), jnp.float32)
  zeros_out = jnp.zeros((N_PAD // NSUB, D_OUT), jnp.float32)

  h = _mm1(x, W0)
  p1 = spmm_hid(h, src_r, dst_r, w_r, zeros_hid)
  h1 = _combine_relu_mm(p1, W1)
  p2 = spmm_out(h1, src_r, dst_r, w_r, zeros_out)
  return _combine_softmax(p2)
